# baseline (speedup 1.0000x reference)
import functools

import jax
import jax.numpy as jnp
from jax import lax
from jax.experimental import pallas as pl
from jax.experimental.pallas import tpu as pltpu
from jax.experimental.pallas import tpu_sc as plsc

_TILE = 512


def _nn_kernel(xyz1_ref, xyz2t_ref, idxw_ref):
    T = xyz1_ref.shape[1]
    N2 = xyz2t_ref.shape[2]

    x1 = xyz1_ref[0]
    x2t = xyz2t_ref[0]

    u0, u1, u2 = x1[:, 0:1], x1[:, 1:2], x1[:, 2:3]
    v0, v1, v2 = x2t[0:1, :], x2t[1:2, :], x2t[2:3, :]
    ab = jnp.dot(x1, x2t, preferred_element_type=jnp.float32)
    a2 = u0 * u0 + u1 * u1 + u2 * u2
    b2 = v0 * v0 + v1 * v1 + v2 * v2
    d2 = a2 + b2 - 2.0 * ab
    dist = jnp.sqrt(jnp.maximum(d2, 0.0))

    fiota = lax.broadcasted_iota(jnp.int32, (T, N2), 1).astype(jnp.float32)
    fN2 = jnp.float32(N2)
    BIG = jnp.float32(3.0e38)
    cur = dist
    idxs, recs = [], []
    for k in range(3):
        m = jnp.min(cur, axis=1, keepdims=True)
        sel_iota = jnp.where(cur == m, fiota, fN2)
        idxk = jnp.min(sel_iota, axis=1, keepdims=True)
        idxs.append(idxk)
        recs.append(1.0 / jnp.maximum(m, 1e-10))
        if k < 2:
            cur = jnp.where(sel_iota == idxk, BIG, cur)
    norm = recs[0] + recs[1] + recs[2]
    ws = [r / norm for r in recs]
    idxw_ref[0] = jnp.concatenate(
        [idxs[0], idxs[1], idxs[2], ws[0], ws[1], ws[2],
         jnp.zeros((T, 2), jnp.float32)], axis=1)


def _make_sc_gather(P, D):
    info = plsc.get_sparse_core_info()
    NC, NS = info.num_cores, info.num_subcores
    NW = NC * NS
    bpw = P // NW
    CH = 128
    NCH = bpw // CH
    mesh = plsc.VectorSubcoreMesh(core_axis_name="c", subcore_axis_name="s")

    @functools.partial(
        pl.kernel, mesh=mesh,
        compiler_params=pltpu.CompilerParams(use_tc_tiling_on_sc=False),
        out_type=jax.ShapeDtypeStruct((P, D), jnp.float32),
        scratch_types=[
            pltpu.VMEM((bpw,), jnp.int32),
            pltpu.VMEM((bpw, D), jnp.float32),
            pltpu.SemaphoreType.DMA,
        ],
    )
    def sc_gather(table_hbm, idx_hbm, out_hbm, idx_v, rows_v, sem):
        wid = lax.axis_index("s") * NC + lax.axis_index("c")
        base = wid * bpw
        pltpu.sync_copy(idx_hbm.at[pl.ds(base, bpw)], idx_v)
        copies = []
        for j in range(NCH):
            copies.append(pltpu.async_copy(
                table_hbm.at[idx_v.at[pl.ds(j * CH, CH)]],
                rows_v.at[pl.ds(j * CH, CH)], sem))
        for c in copies:
            c.wait()
        pltpu.sync_copy(rows_v, out_hbm.at[pl.ds(base, bpw)])

    return sc_gather


def _mlp_kernel(g_ref, idxw_ref, points1_ref,
                W1a_ref, W1b_ref, b1_ref, g1v_ref, beta1_ref,
                W2_ref, b2_ref, g2v_ref, beta2_ref, out_ref):
    C2 = points1_ref.shape[2]
    g = g_ref[0]
    w0 = idxw_ref[0][:, 3:4]
    w1 = idxw_ref[0][:, 4:5]
    w2 = idxw_ref[0][:, 5:6]
    interp = (g[:, 0:C2] * w0 + g[:, C2:2 * C2] * w1
              + g[:, 2 * C2:3 * C2] * w2)

    inv_std = 1.0 / jnp.sqrt(jnp.float32(1.0 + 1e-5))
    x = (jnp.dot(interp, W1a_ref[...], preferred_element_type=jnp.float32)
         + jnp.dot(points1_ref[0], W1b_ref[...],
                   preferred_element_type=jnp.float32))
    x = x + b1_ref[0]
    x = g1v_ref[0] * (x * inv_std) + beta1_ref[0]
    x = jnp.maximum(x, 0.0)

    x = jnp.dot(x, W2_ref[...], preferred_element_type=jnp.float32)
    x = x + b2_ref[0]
    x = g2v_ref[0] * (x * inv_std) + beta2_ref[0]
    x = jnp.maximum(x, 0.0)
    out_ref[0] = x


def kernel(xyz1, xyz2, points1, points2, W1, b1, g1, beta1, W2, b2, g2, beta2):
    B, N1, _ = xyz1.shape
    _, N2, C2 = points2.shape
    C1 = points1.shape[2]
    Cout = W2.shape[1]
    T = _TILE if N1 % _TILE == 0 else N1

    xyz2t = jnp.transpose(xyz2, (0, 2, 1))
    const = lambda b, i: (0, 0)
    per_b = lambda b, i: (b, 0, 0)
    per_tile = lambda b, i: (b, i, 0)

    W1a, W1b = W1[:C2], W1[C2:]
    vecs = [v.reshape(1, -1) for v in (b1, g1, beta1, b2, g2, beta2)]
    b1r, g1r, beta1r, b2r, g2r, beta2r = vecs

    def nn_stage(xyz1_h, xyz2t_h):
        Bh = xyz1_h.shape[0]
        return pl.pallas_call(
            _nn_kernel,
            grid=(Bh, N1 // T),
            in_specs=[
                pl.BlockSpec((1, T, 3), per_tile),
                pl.BlockSpec((1, 3, N2), per_b),
            ],
            out_specs=pl.BlockSpec((1, T, 8), per_tile),
            out_shape=jax.ShapeDtypeStruct((Bh, N1, 8), jnp.float32),
        )(xyz1_h, xyz2t_h)

    def gather_stage(idxw_h, b0, Bh):
        idx3 = idxw_h[..., :3].astype(jnp.int32)
        off = (b0 + jnp.arange(Bh, dtype=jnp.int32)) * N2
        flat_idx = (idx3 + off[:, None, None]).reshape(-1)
        gathered = _make_sc_gather(3 * Bh * N1, C2)(
            points2.reshape(B * N2, C2), flat_idx)
        return gathered.reshape(Bh, N1, 3 * C2)

    def mlp_stage(g3_h, idxw_h, points1_h):
        Bh = g3_h.shape[0]
        return pl.pallas_call(
            _mlp_kernel,
            grid=(Bh, N1 // T),
            in_specs=[
                pl.BlockSpec((1, T, 3 * C2), per_tile),
                pl.BlockSpec((1, T, 8), per_tile),
                pl.BlockSpec((1, T, C1), per_tile),
                pl.BlockSpec(W1a.shape, const),
                pl.BlockSpec(W1b.shape, const),
                pl.BlockSpec((1, Cout), const),
                pl.BlockSpec((1, Cout), const),
                pl.BlockSpec((1, Cout), const),
                pl.BlockSpec(W2.shape, const),
                pl.BlockSpec((1, Cout), const),
                pl.BlockSpec((1, Cout), const),
                pl.BlockSpec((1, Cout), const),
            ],
            out_specs=pl.BlockSpec((1, T, Cout), per_tile),
            out_shape=jax.ShapeDtypeStruct((Bh, N1, Cout), jnp.float32),
        )(g3_h, idxw_h, points1_h,
          W1a, W1b, b1r, g1r, beta1r, W2, b2r, g2r, beta2r)

    if B % 2 == 0:
        H = B // 2
        idxw0 = nn_stage(xyz1[:H], xyz2t[:H])
        g30 = gather_stage(idxw0, 0, H)
        idxw1 = nn_stage(xyz1[H:], xyz2t[H:])
        g31 = gather_stage(idxw1, H, H)
        out0 = mlp_stage(g30, idxw0, points1[:H])
        out1 = mlp_stage(g31, idxw1, points1[H:])
        return jnp.concatenate([out0, out1], axis=0)
    idxw = nn_stage(xyz1, xyz2t)
    g3 = gather_stage(idxw, 0, B)
    return mlp_stage(g3, idxw, points1)


# speedup vs baseline: 1.0391x; 1.0391x over previous
import functools

import jax
import jax.numpy as jnp
from jax import lax
from jax.experimental import pallas as pl
from jax.experimental.pallas import tpu as pltpu
from jax.experimental.pallas import tpu_sc as plsc

_TILE = 512


def _nn_kernel(xyz1_ref, xyz2t_ref, idxw_ref):
    T = xyz1_ref.shape[1]
    N2 = xyz2t_ref.shape[2]

    x1 = xyz1_ref[0]
    x2t = xyz2t_ref[0]

    u0, u1, u2 = x1[:, 0:1], x1[:, 1:2], x1[:, 2:3]
    v0, v1, v2 = x2t[0:1, :], x2t[1:2, :], x2t[2:3, :]
    ab = jnp.dot(x1, x2t, preferred_element_type=jnp.float32)
    a2 = u0 * u0 + u1 * u1 + u2 * u2
    b2 = v0 * v0 + v1 * v1 + v2 * v2
    d2 = a2 + b2 - 2.0 * ab
    dist = jnp.sqrt(jnp.maximum(d2, 0.0))

    fiota = lax.broadcasted_iota(jnp.int32, (T, N2), 1).astype(jnp.float32)
    fN2 = jnp.float32(N2)
    BIG = jnp.float32(3.0e38)
    cur = dist
    idxs, recs = [], []
    for k in range(3):
        m = jnp.min(cur, axis=1, keepdims=True)
        sel_iota = jnp.where(cur == m, fiota, fN2)
        idxk = jnp.min(sel_iota, axis=1, keepdims=True)
        idxs.append(idxk)
        recs.append(1.0 / jnp.maximum(m, 1e-10))
        if k < 2:
            cur = jnp.where(sel_iota == idxk, BIG, cur)
    norm = recs[0] + recs[1] + recs[2]
    ws = [r / norm for r in recs]
    idxw_ref[0] = jnp.concatenate(
        [idxs[0], idxs[1], idxs[2], ws[0], ws[1], ws[2],
         jnp.zeros((T, 2), jnp.float32)], axis=1)


def _make_sc_gather(P, D):
    info = plsc.get_sparse_core_info()
    NC, NS = info.num_cores, info.num_subcores
    NW = NC * NS
    bpw = P // NW
    CH = 128
    NCH = bpw // CH
    mesh = plsc.VectorSubcoreMesh(core_axis_name="c", subcore_axis_name="s")

    @functools.partial(
        pl.kernel, mesh=mesh,
        compiler_params=pltpu.CompilerParams(use_tc_tiling_on_sc=False),
        out_type=jax.ShapeDtypeStruct((P, D), jnp.float32),
        scratch_types=[
            pltpu.VMEM((bpw,), jnp.int32),
            pltpu.VMEM((bpw, D), jnp.float32),
            pltpu.SemaphoreType.DMA,
        ],
    )
    def sc_gather(table_hbm, idx_hbm, out_hbm, idx_v, rows_v, sem):
        wid = lax.axis_index("s") * NC + lax.axis_index("c")
        base = wid * bpw
        pltpu.sync_copy(idx_hbm.at[pl.ds(base, bpw)], idx_v)
        copies = []
        for j in range(NCH):
            copies.append(pltpu.async_copy(
                table_hbm.at[idx_v.at[pl.ds(j * CH, CH)]],
                rows_v.at[pl.ds(j * CH, CH)], sem))
        for c in copies:
            c.wait()
        pltpu.sync_copy(rows_v, out_hbm.at[pl.ds(base, bpw)])

    return sc_gather


def _mlp_kernel(g_ref, idxw_ref, points1_ref,
                W1a_ref, W1b_ref, b1_ref, g1v_ref, beta1_ref,
                W2_ref, b2_ref, g2v_ref, beta2_ref, out_ref):
    C2 = points1_ref.shape[2]
    g = g_ref[0]
    w0 = idxw_ref[0][:, 3:4]
    w1 = idxw_ref[0][:, 4:5]
    w2 = idxw_ref[0][:, 5:6]
    interp = (g[:, 0:C2] * w0 + g[:, C2:2 * C2] * w1
              + g[:, 2 * C2:3 * C2] * w2)

    inv_std = 1.0 / jnp.sqrt(jnp.float32(1.0 + 1e-5))
    x = (jnp.dot(interp, W1a_ref[...], preferred_element_type=jnp.float32)
         + jnp.dot(points1_ref[0], W1b_ref[...],
                   preferred_element_type=jnp.float32))
    x = x + b1_ref[0]
    x = g1v_ref[0] * (x * inv_std) + beta1_ref[0]
    x = jnp.maximum(x, 0.0)

    x = jnp.dot(x, W2_ref[...], preferred_element_type=jnp.float32)
    x = x + b2_ref[0]
    x = g2v_ref[0] * (x * inv_std) + beta2_ref[0]
    x = jnp.maximum(x, 0.0)
    out_ref[0] = x


def kernel(xyz1, xyz2, points1, points2, W1, b1, g1, beta1, W2, b2, g2, beta2):
    B, N1, _ = xyz1.shape
    _, N2, C2 = points2.shape
    C1 = points1.shape[2]
    Cout = W2.shape[1]
    T = _TILE if N1 % _TILE == 0 else N1

    xyz2t = jnp.transpose(xyz2, (0, 2, 1))
    const = lambda b, i: (0, 0)
    per_b = lambda b, i: (b, 0, 0)
    per_tile = lambda b, i: (b, i, 0)

    W1a, W1b = W1[:C2], W1[C2:]
    vecs = [v.reshape(1, -1) for v in (b1, g1, beta1, b2, g2, beta2)]
    b1r, g1r, beta1r, b2r, g2r, beta2r = vecs

    def nn_stage(xyz1_h, xyz2t_h):
        Bh = xyz1_h.shape[0]
        return pl.pallas_call(
            _nn_kernel,
            grid=(Bh, N1 // T),
            in_specs=[
                pl.BlockSpec((1, T, 3), per_tile),
                pl.BlockSpec((1, 3, N2), per_b),
            ],
            out_specs=pl.BlockSpec((1, T, 8), per_tile),
            out_shape=jax.ShapeDtypeStruct((Bh, N1, 8), jnp.float32),
        )(xyz1_h, xyz2t_h)

    def gather_stage(idxw_h, b0, Bh):
        idx3 = idxw_h[..., :3].astype(jnp.int32)
        off = (b0 + jnp.arange(Bh, dtype=jnp.int32)) * N2
        flat_idx = (idx3 + off[:, None, None]).reshape(-1)
        gathered = _make_sc_gather(3 * Bh * N1, C2)(
            points2.reshape(B * N2, C2), flat_idx)
        return gathered.reshape(Bh, N1, 3 * C2)

    def mlp_stage(g3_h, idxw_h, points1_h):
        Bh = g3_h.shape[0]
        return pl.pallas_call(
            _mlp_kernel,
            grid=(Bh, N1 // T),
            in_specs=[
                pl.BlockSpec((1, T, 3 * C2), per_tile),
                pl.BlockSpec((1, T, 8), per_tile),
                pl.BlockSpec((1, T, C1), per_tile),
                pl.BlockSpec(W1a.shape, const),
                pl.BlockSpec(W1b.shape, const),
                pl.BlockSpec((1, Cout), const),
                pl.BlockSpec((1, Cout), const),
                pl.BlockSpec((1, Cout), const),
                pl.BlockSpec(W2.shape, const),
                pl.BlockSpec((1, Cout), const),
                pl.BlockSpec((1, Cout), const),
                pl.BlockSpec((1, Cout), const),
            ],
            out_specs=pl.BlockSpec((1, T, Cout), per_tile),
            out_shape=jax.ShapeDtypeStruct((Bh, N1, Cout), jnp.float32),
        )(g3_h, idxw_h, points1_h,
          W1a, W1b, b1r, g1r, beta1r, W2, b2r, g2r, beta2r)

    idxw = nn_stage(xyz1, xyz2t)
    g3 = gather_stage(idxw, 0, B)
    return mlp_stage(g3, idxw, points1)


# speedup vs baseline: 1.1221x; 1.0799x over previous
import functools

import jax
import jax.numpy as jnp
from jax import lax
from jax.experimental import pallas as pl
from jax.experimental.pallas import tpu as pltpu
from jax.experimental.pallas import tpu_sc as plsc

_TILE = 1024


def _nn_kernel(xyz1_ref, xyz2t_ref, idxw_ref):
    T = xyz1_ref.shape[1]
    N2 = xyz2t_ref.shape[2]

    x1 = xyz1_ref[0]
    x2t = xyz2t_ref[0]

    u0, u1, u2 = x1[:, 0:1], x1[:, 1:2], x1[:, 2:3]
    v0, v1, v2 = x2t[0:1, :], x2t[1:2, :], x2t[2:3, :]
    ab = jnp.dot(x1, x2t, preferred_element_type=jnp.float32)
    a2 = u0 * u0 + u1 * u1 + u2 * u2
    b2 = v0 * v0 + v1 * v1 + v2 * v2
    d2 = a2 + b2 - 2.0 * ab
    dist = jnp.sqrt(jnp.maximum(d2, 0.0))

    fiota = lax.broadcasted_iota(jnp.int32, (T, N2), 1).astype(jnp.float32)
    fN2 = jnp.float32(N2)
    BIG = jnp.float32(3.0e38)
    cur = dist
    idxs, recs = [], []
    for k in range(3):
        m = jnp.min(cur, axis=1, keepdims=True)
        sel_iota = jnp.where(cur == m, fiota, fN2)
        idxk = jnp.min(sel_iota, axis=1, keepdims=True)
        idxs.append(idxk)
        recs.append(1.0 / jnp.maximum(m, 1e-10))
        if k < 2:
            cur = jnp.where(sel_iota == idxk, BIG, cur)
    norm = recs[0] + recs[1] + recs[2]
    ws = [r / norm for r in recs]
    idxw_ref[0] = jnp.concatenate(
        [idxs[0], idxs[1], idxs[2], ws[0], ws[1], ws[2],
         jnp.zeros((T, 2), jnp.float32)], axis=1)


def _make_sc_gather(P, D):
    info = plsc.get_sparse_core_info()
    NC, NS = info.num_cores, info.num_subcores
    NW = NC * NS
    bpw = P // NW
    CH = 128
    NCH = bpw // CH
    mesh = plsc.VectorSubcoreMesh(core_axis_name="c", subcore_axis_name="s")

    @functools.partial(
        pl.kernel, mesh=mesh,
        compiler_params=pltpu.CompilerParams(use_tc_tiling_on_sc=False),
        out_type=jax.ShapeDtypeStruct((P, D), jnp.float32),
        scratch_types=[
            pltpu.VMEM((bpw,), jnp.int32),
            pltpu.VMEM((bpw, D), jnp.float32),
            pltpu.SemaphoreType.DMA,
        ],
    )
    def sc_gather(table_hbm, idx_hbm, out_hbm, idx_v, rows_v, sem):
        wid = lax.axis_index("s") * NC + lax.axis_index("c")
        base = wid * bpw
        pltpu.sync_copy(idx_hbm.at[pl.ds(base, bpw)], idx_v)
        copies = []
        for j in range(NCH):
            copies.append(pltpu.async_copy(
                table_hbm.at[idx_v.at[pl.ds(j * CH, CH)]],
                rows_v.at[pl.ds(j * CH, CH)], sem))
        for c in copies:
            c.wait()
        pltpu.sync_copy(rows_v, out_hbm.at[pl.ds(base, bpw)])

    return sc_gather


def _mlp_kernel(g_ref, idxw_ref, points1_ref,
                W1a_ref, W1b_ref, b1_ref, g1v_ref, beta1_ref,
                W2_ref, b2_ref, g2v_ref, beta2_ref, out_ref):
    C2 = points1_ref.shape[2]
    g = g_ref[0]
    w0 = idxw_ref[0][:, 3:4]
    w1 = idxw_ref[0][:, 4:5]
    w2 = idxw_ref[0][:, 5:6]
    interp = (g[:, 0:C2] * w0 + g[:, C2:2 * C2] * w1
              + g[:, 2 * C2:3 * C2] * w2)

    inv_std = 1.0 / jnp.sqrt(jnp.float32(1.0 + 1e-5))
    x = (jnp.dot(interp, W1a_ref[...], preferred_element_type=jnp.float32)
         + jnp.dot(points1_ref[0], W1b_ref[...],
                   preferred_element_type=jnp.float32))
    x = x + b1_ref[0]
    x = g1v_ref[0] * (x * inv_std) + beta1_ref[0]
    x = jnp.maximum(x, 0.0)

    x = jnp.dot(x, W2_ref[...], preferred_element_type=jnp.float32)
    x = x + b2_ref[0]
    x = g2v_ref[0] * (x * inv_std) + beta2_ref[0]
    x = jnp.maximum(x, 0.0)
    out_ref[0] = x


def kernel(xyz1, xyz2, points1, points2, W1, b1, g1, beta1, W2, b2, g2, beta2):
    B, N1, _ = xyz1.shape
    _, N2, C2 = points2.shape
    C1 = points1.shape[2]
    Cout = W2.shape[1]
    T = _TILE if N1 % _TILE == 0 else N1

    xyz2t = jnp.transpose(xyz2, (0, 2, 1))
    const = lambda b, i: (0, 0)
    per_b = lambda b, i: (b, 0, 0)
    per_tile = lambda b, i: (b, i, 0)

    W1a, W1b = W1[:C2], W1[C2:]
    vecs = [v.reshape(1, -1) for v in (b1, g1, beta1, b2, g2, beta2)]
    b1r, g1r, beta1r, b2r, g2r, beta2r = vecs

    def nn_stage(xyz1_h, xyz2t_h):
        Bh = xyz1_h.shape[0]
        return pl.pallas_call(
            _nn_kernel,
            grid=(Bh, N1 // T),
            in_specs=[
                pl.BlockSpec((1, T, 3), per_tile),
                pl.BlockSpec((1, 3, N2), per_b),
            ],
            out_specs=pl.BlockSpec((1, T, 8), per_tile),
            out_shape=jax.ShapeDtypeStruct((Bh, N1, 8), jnp.float32),
        )(xyz1_h, xyz2t_h)

    def gather_stage(idxw_h, b0, Bh):
        idx3 = idxw_h[..., :3].astype(jnp.int32)
        off = (b0 + jnp.arange(Bh, dtype=jnp.int32)) * N2
        flat_idx = (idx3 + off[:, None, None]).reshape(-1)
        gathered = _make_sc_gather(3 * Bh * N1, C2)(
            points2.reshape(B * N2, C2), flat_idx)
        return gathered.reshape(Bh, N1, 3 * C2)

    def mlp_stage(g3_h, idxw_h, points1_h):
        Bh = g3_h.shape[0]
        return pl.pallas_call(
            _mlp_kernel,
            grid=(Bh, N1 // T),
            in_specs=[
                pl.BlockSpec((1, T, 3 * C2), per_tile),
                pl.BlockSpec((1, T, 8), per_tile),
                pl.BlockSpec((1, T, C1), per_tile),
                pl.BlockSpec(W1a.shape, const),
                pl.BlockSpec(W1b.shape, const),
                pl.BlockSpec((1, Cout), const),
                pl.BlockSpec((1, Cout), const),
                pl.BlockSpec((1, Cout), const),
                pl.BlockSpec(W2.shape, const),
                pl.BlockSpec((1, Cout), const),
                pl.BlockSpec((1, Cout), const),
                pl.BlockSpec((1, Cout), const),
            ],
            out_specs=pl.BlockSpec((1, T, Cout), per_tile),
            out_shape=jax.ShapeDtypeStruct((Bh, N1, Cout), jnp.float32),
        )(g3_h, idxw_h, points1_h,
          W1a, W1b, b1r, g1r, beta1r, W2, b2r, g2r, beta2r)

    idxw = nn_stage(xyz1, xyz2t)
    g3 = gather_stage(idxw, 0, B)
    return mlp_stage(g3, idxw, points1)
